# Initial kernel scaffold; baseline (speedup 1.0000x reference)
import jax
import jax.numpy as jnp
from jax.experimental import pallas as pl


def kernel(hidden_states, assignment, W1, b1, W2, b2):
    raise NotImplementedError("write your pallas kernel here")


# speedup vs baseline: 3.4441x; 3.4441x over previous
import jax
import jax.numpy as jnp
from jax.experimental import pallas as pl
from jax.experimental.pallas import tpu as pltpu


def _ffn_step(meta_ref, x_ref, w1_ref, b1_ref, w2_ref, b2_ref, o_ref):
    x = x_ref[0]
    h = jnp.dot(x, w1_ref[0], preferred_element_type=jnp.float32) + b1_ref[0, 0]
    h = jax.nn.gelu(h)
    y = jnp.dot(h, w2_ref[0], preferred_element_type=jnp.float32) + b2_ref[0, 0]
    o_ref[0] = y


def _moe_ffn(meta, hidden_states, W1, b1, W2, b2):
    B, S, D = hidden_states.shape
    E, _, F = W1.shape

    def x_map(i, m):
        return (m[i], 0, 0)

    def w_map(i, m):
        return (m[B + i], 0, 0)

    def bias_map(i, m):
        return (m[B + i], 0, 0)

    grid_spec = pltpu.PrefetchScalarGridSpec(
        num_scalar_prefetch=1,
        grid=(B,),
        in_specs=[
            pl.BlockSpec((1, S, D), x_map),
            pl.BlockSpec((1, D, F), w_map),
            pl.BlockSpec((1, 1, F), bias_map),
            pl.BlockSpec((1, F, D), w_map),
            pl.BlockSpec((1, 1, D), bias_map),
        ],
        out_specs=pl.BlockSpec((1, S, D), x_map),
    )
    return pl.pallas_call(
        _ffn_step,
        grid_spec=grid_spec,
        out_shape=jax.ShapeDtypeStruct((B, S, D), jnp.float32),
    )(meta, hidden_states, W1, b1[:, None, :], W2, b2[:, None, :])


def kernel(hidden_states, assignment, W1, b1, W2, b2):
    B = hidden_states.shape[0]
    assignment = assignment.astype(jnp.int32)
    order = jnp.argsort(assignment).astype(jnp.int32)
    eid = jnp.take(assignment, order)
    meta = jnp.concatenate([order, eid])
    return _moe_ffn(meta, hidden_states, W1, b1, W2, b2)
